# Initial kernel scaffold; baseline (speedup 1.0000x reference)
import jax
import jax.numpy as jnp
from jax.experimental import pallas as pl


def kernel(x, W1, b1, g1, be1, W2, b2, g2, be2, Wg, att, Wq, bq, Wkv, bkv, Wl1, bl1, Wl2, bl2, Wr1, br1, Wr2, br2):
    raise NotImplementedError("write your pallas kernel here")


# speedup vs baseline: 1097.0491x; 1097.0491x over previous
import functools
import jax
import jax.numpy as jnp
from jax.experimental import pallas as pl
from jax.experimental.pallas import tpu as pltpu

_B, _T, _N, _C = 8, 4, 1024, 11
_D = 64
_HEADS = 4
_K = 16
_HD = _D // _HEADS
_BT = _B * _T
_INT_MAX = 2**31 - 1
_INT_MIN = -(2**31)


def _embed_kernel(xt_ref, w1_ref, b1_ref, g1_ref, be1_ref, w2_ref, b2_ref,
                  g2_ref, be2_ref, ht_ref):
    xt = xt_ref[...]
    h1 = jnp.dot(w1_ref[...], xt, preferred_element_type=jnp.float32)
    h1 = h1 + b1_ref[...].reshape(-1, 1)
    m1 = jnp.mean(h1, axis=1, keepdims=True)
    v1 = jnp.mean((h1 - m1) ** 2, axis=1, keepdims=True)
    h1 = (h1 - m1) / jnp.sqrt(v1 + 1e-5)
    h1 = h1 * g1_ref[...].reshape(-1, 1) + be1_ref[...].reshape(-1, 1)
    h1 = jnp.maximum(h1, 0.0)
    h2 = jnp.dot(w2_ref[...], h1, preferred_element_type=jnp.float32)
    h2 = h2 + b2_ref[...].reshape(-1, 1)
    m2 = jnp.mean(h2, axis=1, keepdims=True)
    v2 = jnp.mean((h2 - m2) ** 2, axis=1, keepdims=True)
    h2 = (h2 - m2) / jnp.sqrt(v2 + 1e-5)
    h2 = h2 * g2_ref[...].reshape(-1, 1) + be2_ref[...].reshape(-1, 1)
    ht_ref[...] = jnp.maximum(h2, 0.0)


def _gat_kernel(pos_ref, ht_ref, wgt_ref, as_ref, an_ref, o_ref):
    pos = pos_ref[0]
    h = jnp.transpose(ht_ref[...])
    hp = jnp.dot(h, wgt_ref[...], preferred_element_type=jnp.float32)
    s1 = jnp.dot(hp, as_ref[...], preferred_element_type=jnp.float32)
    s2t = jax.lax.dot_general(an_ref[...], hp, (((0,), (1,)), ((), ())),
                              preferred_element_type=jnp.float32)
    sqc = jnp.sum(pos * pos, axis=1, keepdims=True)
    gram = jax.lax.dot_general(pos, pos, (((1,), (1,)), ((), ())),
                               preferred_element_type=jnp.float32)
    d2 = jnp.maximum(sqc + jnp.transpose(sqc) - 2.0 * gram, 0.0)
    key = jax.lax.bitcast_convert_type(d2, jnp.int32)
    iota_m = jax.lax.broadcasted_iota(jnp.int32, (_N, _N), 1)
    key = jnp.bitwise_or(jnp.bitwise_and(key, jnp.int32(~1023)), iota_m)
    thr = jnp.full((_N, 1), _INT_MIN, dtype=jnp.int32)
    for _ in range(_K):
        cb = thr + jnp.int32(1 + _INT_MIN)
        m = jnp.min(key - cb, axis=1, keepdims=True)
        thr = m + cb
    sel = key <= thr
    s2mx = jnp.max(s2t, axis=1, keepdims=True)
    for hd in range(_HEADS):
        sc = s1[:, hd:hd + 1] + s2t[hd:hd + 1, :]
        sc = jnp.maximum(sc, 0.2 * sc)
        u = s1[:, hd:hd + 1] + s2mx[hd, 0]
        u = jnp.maximum(u, 0.2 * u)
        p = jnp.where(sel, jnp.exp(sc - u), 0.0)
        rs = jnp.sum(p, axis=1, keepdims=True)
        out_h = jnp.dot(p, hp[:, hd * _HD:(hd + 1) * _HD],
                        preferred_element_type=jnp.float32)
        o_ref[0, :, hd * _HD:(hd + 1) * _HD] = out_h / rs


def _temporal_kernel(o_ref, ht_ref, wqt_ref, bq_ref, wkvt_ref, bkv_ref,
                     wl1t_ref, bl1_ref, wl2t_ref, bl2_ref,
                     wr1t_ref, br1_ref, wr2t_ref, br2_ref, out_ref):
    htb = jnp.transpose(ht_ref[...])
    hs_last = jnp.maximum(o_ref[0, _T - 1] + htb[(_T - 1) * _N:_T * _N], 0.0)
    q = jnp.dot(hs_last, wqt_ref[...], preferred_element_type=jnp.float32)
    q = q + bq_ref[...].reshape(1, -1)
    scale = 1.0 / (float(_D) ** 0.5)
    kms = []
    scs = []
    for t in range(_T):
        hs_t = jnp.maximum(o_ref[0, t] + htb[t * _N:(t + 1) * _N], 0.0)
        km = jnp.dot(hs_t, wkvt_ref[...], preferred_element_type=jnp.float32)
        km = km + bkv_ref[...].reshape(1, -1)
        kms.append(km)
        scs.append(jnp.sum(q * km, axis=1, keepdims=True) * scale)
    mx = jnp.maximum(jnp.maximum(scs[0], scs[1]), jnp.maximum(scs[2], scs[3]))
    es = [jnp.exp(s - mx) for s in scs]
    den = es[0] + es[1] + es[2] + es[3]
    fused = (es[0] * kms[0] + es[1] * kms[1] + es[2] * kms[2]
             + es[3] * kms[3]) / den
    hl = jnp.dot(jnp.maximum(jnp.dot(fused, wl1t_ref[...],
                                     preferred_element_type=jnp.float32)
                             + bl1_ref[...].reshape(1, -1), 0.0),
                 wl2t_ref[...], preferred_element_type=jnp.float32)
    hl = hl + bl2_ref[...].reshape(1, -1)
    hr = jnp.dot(jnp.maximum(jnp.dot(fused, wr1t_ref[...],
                                     preferred_element_type=jnp.float32)
                             + br1_ref[...].reshape(1, -1), 0.0),
                 wr2t_ref[...], preferred_element_type=jnp.float32)
    hr = hr + br2_ref[...].reshape(1, -1)
    sp_l = jax.nn.softplus(hl[:, 2:4]) + 1e-6
    sp_r = jax.nn.softplus(hr[:, 2:4]) + 1e-6
    out = jnp.concatenate([hl[:, 0:2], sp_l, hr[:, 0:2], sp_r], axis=1)
    out_ref[...] = out.reshape(1, _N, 8)


@functools.partial(jax.jit, static_argnames=("interpret",))
def _run(x, W1, b1, g1, be1, W2, b2, g2, be2, Wg, att, Wq, bq, Wkv, bkv,
         Wl1, bl1, Wl2, bl2, Wr1, br1, Wr2, br2, interpret=False):
    xt = x.reshape(_BT * _N, _C).T
    ht = pl.pallas_call(
        _embed_kernel,
        out_shape=jax.ShapeDtypeStruct((_D, _BT * _N), jnp.float32),
        interpret=interpret,
    )(xt, W1, b1, g1, be1, W2, b2, g2, be2)

    a_self = att[0, :, :_HD]
    a_nei = att[0, :, _HD:]
    eye = jnp.eye(_HEADS, dtype=jnp.float32)
    As = (a_self[:, :, None] * eye[:, None, :]).reshape(_D, _HEADS)
    An = (a_nei[:, :, None] * eye[:, None, :]).reshape(_D, _HEADS)

    pos = x.reshape(_BT, _N, _C)[:, :, :2]
    o = pl.pallas_call(
        _gat_kernel,
        grid=(_BT,),
        in_specs=[
            pl.BlockSpec((1, _N, 2), lambda i: (i, 0, 0)),
            pl.BlockSpec((_D, _N), lambda i: (0, i)),
            pl.BlockSpec((_D, _D), lambda i: (0, 0)),
            pl.BlockSpec((_D, _HEADS), lambda i: (0, 0)),
            pl.BlockSpec((_D, _HEADS), lambda i: (0, 0)),
        ],
        out_specs=pl.BlockSpec((1, _N, _D), lambda i: (i, 0, 0)),
        out_shape=jax.ShapeDtypeStruct((_BT, _N, _D), jnp.float32),
        compiler_params=pltpu.CompilerParams(
            dimension_semantics=("arbitrary",)),
        interpret=interpret,
    )(pos, ht, Wg.T, As, An)

    out8 = pl.pallas_call(
        _temporal_kernel,
        grid=(_B,),
        in_specs=[
            pl.BlockSpec((1, _T, _N, _D), lambda b: (b, 0, 0, 0)),
            pl.BlockSpec((_D, _T * _N), lambda b: (0, b)),
            pl.BlockSpec((_D, _D), lambda b: (0, 0)),
            pl.BlockSpec((_D,), lambda b: (0,)),
            pl.BlockSpec((_D, _D), lambda b: (0, 0)),
            pl.BlockSpec((_D,), lambda b: (0,)),
            pl.BlockSpec((_D, 32), lambda b: (0, 0)),
            pl.BlockSpec((32,), lambda b: (0,)),
            pl.BlockSpec((32, 4), lambda b: (0, 0)),
            pl.BlockSpec((4,), lambda b: (0,)),
            pl.BlockSpec((_D, 32), lambda b: (0, 0)),
            pl.BlockSpec((32,), lambda b: (0,)),
            pl.BlockSpec((32, 4), lambda b: (0, 0)),
            pl.BlockSpec((4,), lambda b: (0,)),
        ],
        out_specs=pl.BlockSpec((1, _N, 8), lambda b: (b, 0, 0)),
        out_shape=jax.ShapeDtypeStruct((_B, _N, 8), jnp.float32),
        compiler_params=pltpu.CompilerParams(
            dimension_semantics=("arbitrary",)),
        interpret=interpret,
    )(o.reshape(_B, _T, _N, _D), ht,
      Wq.T, bq, Wkv.T, bkv, Wl1.T, bl1, Wl2.T, bl2, Wr1.T, br1, Wr2.T, br2)

    return (out8[:, :, 0:2], out8[:, :, 2:4],
            out8[:, :, 4:6], out8[:, :, 6:8])


def kernel(x, W1, b1, g1, be1, W2, b2, g2, be2, Wg, att, Wq, bq, Wkv, bkv,
           Wl1, bl1, Wl2, bl2, Wr1, br1, Wr2, br2):
    return _run(x, W1, b1, g1, be1, W2, b2, g2, be2, Wg, att, Wq, bq,
                Wkv, bkv, Wl1, bl1, Wl2, bl2, Wr1, br1, Wr2, br2)
